# baseline (speedup 1.0000x reference)
import functools

import jax
import jax.numpy as jnp
from jax import lax
from jax.experimental import pallas as pl
from jax.experimental.pallas import tpu as pltpu, tpu_sc as plsc

EMBED = 64
BATCH = 16384
NC, NS, L = 2, 16, 16
NW = NC * NS
B_PER_W = BATCH // NW
CH = 128
NCHUNK = B_PER_W // CH


def _body(ph_i, pt_i, pr_i, nh_i, nt_i, nr_i, ent, rel,
          o_pos, o_neg, o_ph, o_pt, o_nh, o_nt,
          idx, b_ph, b_pt, b_pr, b_nh, b_nt, b_nr, sem):
    wid = lax.axis_index("s") * NC + lax.axis_index("c")
    pltpu.sync_copy(ph_i.at[wid], idx.at[0])
    pltpu.sync_copy(pt_i.at[wid], idx.at[1])
    pltpu.sync_copy(pr_i.at[wid], idx.at[2])
    pltpu.sync_copy(nh_i.at[wid], idx.at[3])
    pltpu.sync_copy(nt_i.at[wid], idx.at[4])
    pltpu.sync_copy(nr_i.at[wid], idx.at[5])

    for c in range(NCHUNK):
        row0 = wid * B_PER_W + c * CH
        gathers = [
            pltpu.async_copy(ent.at[idx.at[0, c]], b_ph, sem),
            pltpu.async_copy(ent.at[idx.at[1, c]], b_pt, sem),
            pltpu.async_copy(rel.at[idx.at[2, c]], b_pr, sem),
            pltpu.async_copy(ent.at[idx.at[3, c]], b_nh, sem),
            pltpu.async_copy(ent.at[idx.at[4, c]], b_nt, sem),
            pltpu.async_copy(rel.at[idx.at[5, c]], b_nr, sem),
        ]
        for g in gathers:
            g.wait()

        def compute(i, _):
            for j in range(EMBED // L):
                sl = pl.ds(j * L, L)
                b_pr[i, sl] = b_ph[i, sl] + b_pr[i, sl] - b_pt[i, sl]
                b_nr[i, sl] = b_nh[i, sl] + b_nr[i, sl] - b_nt[i, sl]
            return ()

        lax.fori_loop(0, CH, compute, ())

        pltpu.sync_copy(b_pr, o_pos.at[pl.ds(row0, CH)])
        pltpu.sync_copy(b_nr, o_neg.at[pl.ds(row0, CH)])
        pltpu.sync_copy(b_ph, o_ph.at[pl.ds(row0, CH)])
        pltpu.sync_copy(b_pt, o_pt.at[pl.ds(row0, CH)])
        pltpu.sync_copy(b_nh, o_nh.at[pl.ds(row0, CH)])
        pltpu.sync_copy(b_nt, o_nt.at[pl.ds(row0, CH)])


def kernel(pos_h, pos_t, pos_r, neg_h, neg_t, neg_r, ent_emb, rel_emb):
    def shaped(x):
        return x.astype(jnp.int32).reshape(NW, NCHUNK, CH)

    out = jax.ShapeDtypeStruct((BATCH, EMBED), jnp.float32)
    run = pl.kernel(
        _body,
        out_type=(out,) * 6,
        mesh=plsc.VectorSubcoreMesh(core_axis_name="c", subcore_axis_name="s"),
        scratch_types=[
            pltpu.VMEM((6, NCHUNK, CH), jnp.int32),
        ] + [pltpu.VMEM((CH, EMBED), jnp.float32)] * 6 + [
            pltpu.SemaphoreType.DMA,
        ],
        compiler_params=pltpu.CompilerParams(use_tc_tiling_on_sc=False),
    )
    return run(shaped(pos_h), shaped(pos_t), shaped(pos_r),
               shaped(neg_h), shaped(neg_t), shaped(neg_r),
               ent_emb, rel_emb)


# speedup vs baseline: 1.0092x; 1.0092x over previous
import jax
import jax.numpy as jnp
from jax import lax
from jax.experimental import pallas as pl
from jax.experimental.pallas import tpu as pltpu, tpu_sc as plsc

EMBED = 64
BATCH = 16384
ENT_ROWS = 1000000
REL_ROWS = 1000
NC, NS, L = 2, 16, 16
NW = NC * NS
B_PER_W = BATCH // NW
CH = 64
NCHUNK = B_PER_W // CH
W2 = 2 * EMBED


def _body(idx_hbm, ent, rel, o_pos, o_neg, o_ph, o_pt, o_nh, o_nt,
          idxv, b0, b1, b2, b3, b4, b5, c0, c1, c2, c3, c4, c5,
          sem_g, sem_o):
    wid = lax.axis_index("s") * NC + lax.axis_index("c")
    pltpu.sync_copy(idx_hbm.at[:, pl.ds(wid * B_PER_W, B_PER_W)],
                    idxv.at[:, pl.ds(0, B_PER_W)])

    sets = ((b0, b1, b2, b3, b4, b5), (c0, c1, c2, c3, c4, c5))
    tables = (ent, ent, rel, ent, ent, rel)
    outs = (o_ph, o_pt, o_pos, o_nh, o_nt, o_neg)

    def issue_gathers(c):
        bufs = sets[c % 2]
        return [
            pltpu.async_copy(tables[k].at[idxv.at[k, pl.ds(c * CH, CH)]],
                             bufs[k], sem_g)
            for k in range(6)
        ]

    gh = {0: issue_gathers(0)}
    oh = {}
    for c in range(NCHUNK):
        bufs = sets[c % 2]
        b_ph, b_pt, b_pr, b_nh, b_nt, b_nr = bufs
        for g in gh.pop(c):
            g.wait()
        if c - 1 in oh:
            for o in oh.pop(c - 1):
                o.wait()
        if c + 1 < NCHUNK:
            gh[c + 1] = issue_gathers(c + 1)

        col0 = c * CH

        def compute(g, _):
            base = pl.multiple_of(col0 + g * L, L)
            offs = [idxv[r, pl.ds(base, L)] for r in range(6, 12)]
            for k in range(L):
                i = g * L + k
                s = [pl.multiple_of(offs[m][k], L) for m in range(6)]
                for j in range(EMBED // L):
                    lo = j * L
                    vph = b_ph[i, pl.ds(s[0] + lo, L)]
                    vpt = b_pt[i, pl.ds(s[1] + lo, L)]
                    vpr = b_pr[i, pl.ds(s[2] + lo, L)]
                    vnh = b_nh[i, pl.ds(s[3] + lo, L)]
                    vnt = b_nt[i, pl.ds(s[4] + lo, L)]
                    vnr = b_nr[i, pl.ds(s[5] + lo, L)]
                    sl = pl.ds(lo, L)
                    b_ph[i, sl] = vph
                    b_pt[i, sl] = vpt
                    b_nh[i, sl] = vnh
                    b_nt[i, sl] = vnt
                    b_pr[i, sl] = vph + vpr - vpt
                    b_nr[i, sl] = vnh + vnr - vnt
            return ()

        lax.fori_loop(0, CH // L, compute, ())

        row0 = wid * B_PER_W + col0
        oh[c] = [
            pltpu.async_copy(bufs[k], outs[k].at[pl.ds(row0, CH)], sem_o)
            for k in range(6)
        ]
    for hs in oh.values():
        for o in hs:
            o.wait()


def kernel(pos_h, pos_t, pos_r, neg_h, neg_t, neg_r, ent_emb, rel_emb):
    ent2 = ent_emb.reshape(ENT_ROWS // 2, W2)
    rel2 = rel_emb.reshape(REL_ROWS // 2, W2)
    idx32 = [x.astype(jnp.int32)
             for x in (pos_h, pos_t, pos_r, neg_h, neg_t, neg_r)]
    idx_all = jnp.stack([x >> 1 for x in idx32]
                        + [(x & 1) << 6 for x in idx32])

    out = jax.ShapeDtypeStruct((BATCH, W2), jnp.float32)
    run = pl.kernel(
        _body,
        out_type=(out,) * 6,
        mesh=plsc.VectorSubcoreMesh(core_axis_name="c", subcore_axis_name="s"),
        scratch_types=[
            pltpu.VMEM((12, B_PER_W + L), jnp.int32),
        ] + [pltpu.VMEM((CH, W2), jnp.float32)] * 12 + [
            pltpu.SemaphoreType.DMA,
            pltpu.SemaphoreType.DMA,
        ],
    )
    o_pos, o_neg, o_ph, o_pt, o_nh, o_nt = run(idx_all, ent2, rel2)
    return (o_pos[:, :EMBED], o_neg[:, :EMBED], o_ph[:, :EMBED],
            o_pt[:, :EMBED], o_nh[:, :EMBED], o_nt[:, :EMBED])


# speedup vs baseline: 1.0102x; 1.0010x over previous
import jax
import jax.numpy as jnp
from jax import lax
from jax.experimental import pallas as pl
from jax.experimental.pallas import tpu as pltpu, tpu_sc as plsc

EMBED = 64
BATCH = 16384
ENT_ROWS = 1000000
REL_ROWS = 1000
NC, NS, L = 2, 16, 16
NW = NC * NS
B_PER_W = BATCH // NW
CH = 64
NCHUNK = B_PER_W // CH
W2 = 2 * EMBED


def _body(idx_hbm, ent, rel, o_pos, o_neg, o_ph, o_pt, o_nh, o_nt,
          idxv, b0, b1, b2, b3, b4, b5, c0, c1, c2, c3, c4, c5,
          sem_g, sem_o):
    wid = lax.axis_index("s") * NC + lax.axis_index("c")
    pltpu.sync_copy(idx_hbm.at[:, pl.ds(wid * B_PER_W, B_PER_W)],
                    idxv.at[:, pl.ds(0, B_PER_W)])

    sets = ((b0, b1, b2, b3, b4, b5), (c0, c1, c2, c3, c4, c5))
    tables = (ent, ent, rel, ent, ent, rel)
    outs = (o_ph, o_pt, o_pos, o_nh, o_nt, o_neg)

    def issue_gathers(c):
        bufs = sets[c % 2]
        return [
            pltpu.async_copy(tables[k].at[idxv.at[k, pl.ds(c * CH, CH)]],
                             bufs[k], sem_g)
            for k in range(6)
        ]

    gh = {0: issue_gathers(0)}
    oh = {}
    for c in range(NCHUNK):
        bufs = sets[c % 2]
        b_ph, b_pt, b_pr, b_nh, b_nt, b_nr = bufs
        for g in gh.pop(c):
            g.wait()
        if c - 1 in oh:
            for o in oh.pop(c - 1):
                o.wait()
        if c + 1 < NCHUNK:
            gh[c + 1] = issue_gathers(c + 1)

        col0 = c * CH

        def compute(g, _):
            base = pl.multiple_of(col0 + g * L, L)
            offs = [idxv[r, pl.ds(base, L)] for r in range(6, 12)]
            for k in range(L):
                i = g * L + k
                s = [pl.multiple_of(offs[m][k], L) for m in range(6)]
                for j in range(EMBED // L):
                    lo = j * L
                    vph = b_ph[i, pl.ds(s[0] + lo, L)]
                    vpt = b_pt[i, pl.ds(s[1] + lo, L)]
                    vpr = b_pr[i, pl.ds(s[2] + lo, L)]
                    vnh = b_nh[i, pl.ds(s[3] + lo, L)]
                    vnt = b_nt[i, pl.ds(s[4] + lo, L)]
                    vnr = b_nr[i, pl.ds(s[5] + lo, L)]
                    sl = pl.ds(lo, L)
                    b_ph[i, sl] = vph
                    b_pt[i, sl] = vpt
                    b_nh[i, sl] = vnh
                    b_nt[i, sl] = vnt
                    b_pr[i, sl] = vph + vpr - vpt
                    b_nr[i, sl] = vnh + vnr - vnt
            return ()

        lax.fori_loop(0, CH // L, compute, ())

        row0 = wid * B_PER_W + col0
        oh[c] = [
            pltpu.async_copy(bufs[k], outs[k].at[pl.ds(row0, CH)], sem_o)
            for k in range(6)
        ]
    for hs in oh.values():
        for o in hs:
            o.wait()


def kernel(pos_h, pos_t, pos_r, neg_h, neg_t, neg_r, ent_emb, rel_emb):
    ent2 = ent_emb.reshape(ENT_ROWS // 2, W2)
    rel2 = rel_emb.reshape(REL_ROWS // 2, W2)
    idx32 = [x.astype(jnp.int32)
             for x in (pos_h, pos_t, pos_r, neg_h, neg_t, neg_r)]
    idx_all = jnp.stack([x >> 1 for x in idx32]
                        + [(x & 1) << 6 for x in idx32])

    out = jax.ShapeDtypeStruct((BATCH, W2), jnp.float32)
    run = pl.kernel(
        _body,
        out_type=(out,) * 6,
        mesh=plsc.VectorSubcoreMesh(core_axis_name="c", subcore_axis_name="s"),
        scratch_types=[
            pltpu.VMEM((12, B_PER_W + L), jnp.int32),
        ] + [pltpu.VMEM((CH, W2), jnp.float32)] * 12 + [
            pltpu.SemaphoreType.DMA,
            pltpu.SemaphoreType.DMA,
        ],
        compiler_params=pltpu.CompilerParams(use_tc_tiling_on_sc=False),
    )
    o_pos, o_neg, o_ph, o_pt, o_nh, o_nt = run(idx_all, ent2, rel2)
    return (o_pos[:, :EMBED], o_neg[:, :EMBED], o_ph[:, :EMBED],
            o_pt[:, :EMBED], o_nh[:, :EMBED], o_nt[:, :EMBED])
